# baseline (speedup 1.0000x reference)
import functools

import jax
import jax.numpy as jnp
from jax import lax
from jax.experimental import pallas as pl
from jax.experimental.pallas import tpu as pltpu
from jax.experimental.pallas import tpu_sc as plsc

N = 10000
E = 320000
D = 128
BN_EPS = 1e-5

NUM_TILES = 16
CHUNK = 128
CEDGES = CHUNK
NCHUNK = 160
TILE_EDGES = NCHUNK * CEDGES
E_PAD = NUM_TILES * TILE_EDGES
N_PAD = N + 8
ROWS_PER_TILE = 624
ROWS_TAIL = N - NUM_TILES * ROWS_PER_TILE


def _sc_aggregate(x_pad, edges):
    mesh = plsc.VectorSubcoreMesh(core_axis_name="c", subcore_axis_name="s")

    @functools.partial(
        pl.kernel,
        out_type=jax.ShapeDtypeStruct((2, N, D), jnp.float32),
        mesh=mesh,
        scratch_types=[
            pltpu.VMEM_SHARED((N_PAD, D), jnp.float32),
            pltpu.VMEM((2, 2, CEDGES), jnp.int32),
            pltpu.VMEM((CEDGES, D), jnp.float32),
            pltpu.SemaphoreType.DMA((2,)),
            pltpu.SemaphoreType.DMA,
            pltpu.SemaphoreType.DMA,
        ],
    )
    def agg_kernel(x_hbm, edges_hbm, out_hbm, acc, idx, rows, isem, gsem, ssem):
        c = lax.axis_index("c")
        s = lax.axis_index("s")
        r0 = pl.multiple_of(s * ROWS_PER_TILE, 8)

        def idx_start(j, t):
            pltpu.async_copy(edges_hbm.at[c, s, j], idx.at[t], isem.at[t])

        def idx_wait(j, t):
            pltpu.make_async_copy(
                edges_hbm.at[c, s, j], idx.at[t], isem.at[t]).wait()

        idx_start(0, 0)
        pltpu.sync_copy(x_hbm.at[pl.ds(r0, ROWS_PER_TILE)],
                        acc.at[pl.ds(r0, ROWS_PER_TILE)])

        @pl.when(s == NUM_TILES - 1)
        def _init_tail():
            t0 = NUM_TILES * ROWS_PER_TILE
            pltpu.sync_copy(x_hbm.at[pl.ds(t0, N_PAD - t0)],
                            acc.at[pl.ds(t0, N_PAD - t0)])

        plsc.subcore_barrier()

        def step(j, t, prefetch):
            idx_wait(j, t)
            if prefetch:
                idx_start(j + 1, 1 - t)
            pltpu.async_copy(x_hbm.at[idx.at[t, 0]], rows, gsem).wait()
            pltpu.async_copy(rows, acc.at[idx.at[t, 1]], ssem, add=True)
            pltpu.make_async_copy(rows, acc.at[idx.at[t, 1]], ssem).wait()

        def outer(jo, carry):
            step(jo * 2, 0, prefetch=True)
            step(jo * 2 + 1, 1, prefetch=True)
            return carry

        lax.fori_loop(0, NCHUNK // 2 - 1, outer, 0)
        step(NCHUNK - 2, 0, prefetch=True)
        step(NCHUNK - 1, 1, prefetch=False)

        plsc.subcore_barrier()

        pltpu.sync_copy(acc.at[pl.ds(r0, ROWS_PER_TILE)],
                        out_hbm.at[c, pl.ds(r0, ROWS_PER_TILE)])

        @pl.when(s == NUM_TILES - 1)
        def _out_tail():
            t0 = NUM_TILES * ROWS_PER_TILE
            pltpu.sync_copy(acc.at[pl.ds(t0, ROWS_TAIL)],
                            out_hbm.at[c, pl.ds(t0, ROWS_TAIL)])

    return agg_kernel(x_pad, edges)


def _tc_mlp(pre, w1t0, w2t0, g10, b10, g20, b20, w1t1, w2t1, g11, b11, g21, b21):
    def body(pre_ref, w1t0_r, w2t0_r, g10_r, b10_r, g20_r, b20_r,
             w1t1_r, w2t1_r, g11_r, b11_r, g21_r, b21_r, out_ref):
        def bn_relu(h, g, b):
            mean = jnp.mean(h, axis=0, keepdims=True)
            var = jnp.mean((h - mean) * (h - mean), axis=0, keepdims=True)
            return jnp.maximum((h - mean) * lax.rsqrt(var + BN_EPS) * g + b, 0.0)

        def rel(p, w1t, w2t, g1, b1, g2, b2):
            h = jnp.dot(p, w1t, preferred_element_type=jnp.float32)
            h = bn_relu(h, g1, b1)
            h = jnp.dot(h, w2t, preferred_element_type=jnp.float32)
            return bn_relu(h, g2, b2)

        out_ref[...] = (
            rel(pre_ref[0], w1t0_r[...], w2t0_r[...], g10_r[...], b10_r[...],
                g20_r[...], b20_r[...])
            + rel(pre_ref[1], w1t1_r[...], w2t1_r[...], g11_r[...], b11_r[...],
                  g21_r[...], b21_r[...]))

    return pl.pallas_call(
        body,
        out_shape=jax.ShapeDtypeStruct((N, D), jnp.float32),
    )(pre, w1t0, w2t0, g10, b10, g20, b20, w1t1, w2t1, g11, b11, g21, b21)


def _prep_edges(edge_index):
    pad = E_PAD - E
    src = jnp.concatenate([edge_index[0], jnp.full((pad,), N, jnp.int32)])
    dst = jnp.concatenate(
        [edge_index[1], (jnp.arange(pad, dtype=jnp.int32) * 37) % N])
    src = src.reshape(NUM_TILES, NCHUNK, 1, CEDGES)
    dst = dst.reshape(NUM_TILES, NCHUNK, 1, CEDGES)
    return jnp.concatenate([src, dst], axis=2)


@jax.jit
def kernel(x, edge_index_rel0, edge_index_rel1,
           W1_0, W2_0, g1_0, b1_0, g2_0, b2_0,
           W1_1, W2_1, g1_1, b1_1, g2_1, b2_1):
    edges = jnp.stack([_prep_edges(edge_index_rel0),
                       _prep_edges(edge_index_rel1)])
    x_pad = jnp.concatenate([x, jnp.zeros((N_PAD - N, D), jnp.float32)])
    pre = _sc_aggregate(x_pad, edges)
    row = lambda v: v.reshape(1, D)
    return _tc_mlp(pre,
                   W1_0.T, W2_0.T, row(g1_0), row(b1_0), row(g2_0), row(b2_0),
                   W1_1.T, W2_1.T, row(g1_1), row(b1_1), row(g2_1), row(b2_1))


# speedup vs baseline: 2.4172x; 2.4172x over previous
import functools

import jax
import jax.numpy as jnp
from jax import lax
from jax.experimental import pallas as pl
from jax.experimental.pallas import tpu as pltpu
from jax.experimental.pallas import tpu_sc as plsc

N = 10000
E = 320000
D = 128
BN_EPS = 1e-5

NUM_TILES = 16
EDGES_PER_TILE = E // NUM_TILES
CHUNK = 128
NFULL = EDGES_PER_TILE // CHUNK
REM = EDGES_PER_TILE - NFULL * CHUNK
ROWS_PER_TILE = 624
ROWS_TAIL = N - NUM_TILES * ROWS_PER_TILE


def _sc_aggregate(x, edges):
    mesh = plsc.VectorSubcoreMesh(core_axis_name="c", subcore_axis_name="s")

    @functools.partial(
        pl.kernel,
        out_type=jax.ShapeDtypeStruct((2, N, D), jnp.float32),
        mesh=mesh,
        scratch_types=[
            pltpu.VMEM_SHARED((N, D), jnp.float32),
            pltpu.VMEM((CHUNK,), jnp.int32),
            pltpu.VMEM((CHUNK,), jnp.int32),
            pltpu.VMEM((CHUNK,), jnp.int32),
            pltpu.VMEM((CHUNK,), jnp.int32),
            pltpu.VMEM((CHUNK, D), jnp.float32),
            pltpu.VMEM((REM,), jnp.int32),
            pltpu.VMEM((REM,), jnp.int32),
            pltpu.VMEM((REM, D), jnp.float32),
            pltpu.SemaphoreType.DMA,
            pltpu.SemaphoreType.DMA,
            pltpu.SemaphoreType.DMA,
            pltpu.SemaphoreType.DMA,
            pltpu.SemaphoreType.DMA,
        ],
    )
    def agg_kernel(x_hbm, edges_hbm, out_hbm, acc, src_a, dst_a, src_b, dst_b,
                   rows_v, src_r, dst_r, rows_r, sem, sis_a, sid_a, sis_b,
                   sid_b):
        c = lax.axis_index("c")
        s = lax.axis_index("s")
        r0 = pl.multiple_of(s * ROWS_PER_TILE, 8)
        base = pl.multiple_of(c * (2 * E) + s * EDGES_PER_TILE, 8)

        bufs = ((src_a, dst_a, sis_a, sid_a), (src_b, dst_b, sis_b, sid_b))

        def idx_start(j, p):
            sv, dv, ss, sd = bufs[p]
            off = pl.multiple_of(base + j * CHUNK, 8)
            pltpu.async_copy(edges_hbm.at[pl.ds(off, CHUNK)], sv, ss)
            pltpu.async_copy(edges_hbm.at[pl.ds(off + E, CHUNK)], dv, sd)

        def idx_wait(j, p):
            sv, dv, ss, sd = bufs[p]
            off = pl.multiple_of(base + j * CHUNK, 8)
            pltpu.make_async_copy(
                edges_hbm.at[pl.ds(off, CHUNK)], sv, ss).wait()
            pltpu.make_async_copy(
                edges_hbm.at[pl.ds(off + E, CHUNK)], dv, sd).wait()

        idx_start(0, 0)
        pltpu.sync_copy(x_hbm.at[pl.ds(r0, ROWS_PER_TILE)],
                        acc.at[pl.ds(r0, ROWS_PER_TILE)])

        @pl.when(s == NUM_TILES - 1)
        def _init_tail():
            t0 = NUM_TILES * ROWS_PER_TILE
            pltpu.sync_copy(x_hbm.at[pl.ds(t0, ROWS_TAIL)],
                            acc.at[pl.ds(t0, ROWS_TAIL)])

        plsc.subcore_barrier()

        def step(j, p, prefetch):
            sv, dv, _, _ = bufs[p]
            idx_wait(j, p)
            if prefetch:
                idx_start(j + 1, 1 - p)
            pltpu.async_copy(x_hbm.at[sv], rows_v, sem).wait()
            pltpu.sync_copy(rows_v, acc.at[dv], add=True)

        def body(jo, carry):
            step(jo * 2, 0, prefetch=True)
            step(jo * 2 + 1, 1, prefetch=True)
            return carry

        lax.fori_loop(0, NFULL // 2 - 1, body, 0)
        step(NFULL - 2, 0, prefetch=True)
        step(NFULL - 1, 1, prefetch=False)

        off = base + NFULL * CHUNK
        pltpu.sync_copy(edges_hbm.at[pl.ds(off, REM)], src_r)
        pltpu.sync_copy(edges_hbm.at[pl.ds(off + E, REM)], dst_r)
        pltpu.async_copy(x_hbm.at[src_r], rows_r, sem).wait()
        pltpu.sync_copy(rows_r, acc.at[dst_r], add=True)

        plsc.subcore_barrier()
        pltpu.sync_copy(acc.at[pl.ds(r0, ROWS_PER_TILE)],
                        out_hbm.at[c, pl.ds(r0, ROWS_PER_TILE)])

        @pl.when(s == NUM_TILES - 1)
        def _out_tail():
            t0 = NUM_TILES * ROWS_PER_TILE
            pltpu.sync_copy(acc.at[pl.ds(t0, ROWS_TAIL)],
                            out_hbm.at[c, pl.ds(t0, ROWS_TAIL)])

    return agg_kernel(x, edges)


def _tc_mlp(pre, w1t0, w2t0, g10, b10, g20, b20, w1t1, w2t1, g11, b11, g21, b21):
    def body(pre_ref, w1t0_r, w2t0_r, g10_r, b10_r, g20_r, b20_r,
             w1t1_r, w2t1_r, g11_r, b11_r, g21_r, b21_r, out_ref):
        def bn_relu(h, g, b):
            mean = jnp.mean(h, axis=0, keepdims=True)
            var = jnp.mean((h - mean) * (h - mean), axis=0, keepdims=True)
            return jnp.maximum((h - mean) * lax.rsqrt(var + BN_EPS) * g + b, 0.0)

        def rel(p, w1t, w2t, g1, b1, g2, b2):
            h = jnp.dot(p, w1t, preferred_element_type=jnp.float32)
            h = bn_relu(h, g1, b1)
            h = jnp.dot(h, w2t, preferred_element_type=jnp.float32)
            return bn_relu(h, g2, b2)

        out_ref[...] = (
            rel(pre_ref[0], w1t0_r[...], w2t0_r[...], g10_r[...], b10_r[...],
                g20_r[...], b20_r[...])
            + rel(pre_ref[1], w1t1_r[...], w2t1_r[...], g11_r[...], b11_r[...],
                  g21_r[...], b21_r[...]))

    return pl.pallas_call(
        body,
        out_shape=jax.ShapeDtypeStruct((N, D), jnp.float32),
    )(pre, w1t0, w2t0, g10, b10, g20, b20, w1t1, w2t1, g11, b11, g21, b21)


@jax.jit
def kernel(x, edge_index_rel0, edge_index_rel1,
           W1_0, W2_0, g1_0, b1_0, g2_0, b2_0,
           W1_1, W2_1, g1_1, b1_1, g2_1, b2_1):
    edges = jnp.concatenate(
        [edge_index_rel0.reshape(-1), edge_index_rel1.reshape(-1)])
    pre = _sc_aggregate(x, edges)
    row = lambda v: v.reshape(1, D)
    return _tc_mlp(pre,
                   W1_0.T, W2_0.T, row(g1_0), row(b1_0), row(g2_0), row(b2_0),
                   W1_1.T, W2_1.T, row(g1_1), row(b1_1), row(g2_1), row(b2_1))


# speedup vs baseline: 2.5793x; 1.0671x over previous
import functools

import jax
import jax.numpy as jnp
from jax import lax
from jax.experimental import pallas as pl
from jax.experimental.pallas import tpu as pltpu
from jax.experimental.pallas import tpu_sc as plsc

N = 10000
E = 320000
D = 128
BN_EPS = 1e-5

NUM_TILES = 16
EDGES_PER_TILE = E // NUM_TILES
CHUNK = 128
NFULL = EDGES_PER_TILE // CHUNK
REM = EDGES_PER_TILE - NFULL * CHUNK
ROWS_PER_TILE = 624
ROWS_TAIL = N - NUM_TILES * ROWS_PER_TILE


def _sc_aggregate(x, edges):
    mesh = plsc.VectorSubcoreMesh(core_axis_name="c", subcore_axis_name="s")

    @functools.partial(
        pl.kernel,
        out_type=jax.ShapeDtypeStruct((2, N, D), jnp.float32),
        mesh=mesh,
        scratch_types=[
            pltpu.VMEM_SHARED((N, D), jnp.float32),
            pltpu.VMEM((CHUNK,), jnp.int32),
            pltpu.VMEM((CHUNK,), jnp.int32),
            pltpu.VMEM((CHUNK,), jnp.int32),
            pltpu.VMEM((CHUNK,), jnp.int32),
            pltpu.VMEM((CHUNK, D), jnp.float32),
            pltpu.VMEM((CHUNK, D), jnp.float32),
            pltpu.VMEM((REM,), jnp.int32),
            pltpu.VMEM((REM,), jnp.int32),
            pltpu.VMEM((REM, D), jnp.float32),
            pltpu.SemaphoreType.DMA,
            pltpu.SemaphoreType.DMA,
            pltpu.SemaphoreType.DMA,
            pltpu.SemaphoreType.DMA,
            pltpu.SemaphoreType.DMA,
            pltpu.SemaphoreType.DMA,
            pltpu.SemaphoreType.DMA,
        ],
    )
    def agg_kernel(x_hbm, edges_hbm, out_hbm, acc, src_a, dst_a, src_b, dst_b,
                   rows_a, rows_b, src_r, dst_r, rows_r, sem, sis_a, sid_a,
                   sis_b, sid_b, ss_a, ss_b):
        c = lax.axis_index("c")
        s = lax.axis_index("s")
        r0 = pl.multiple_of(s * ROWS_PER_TILE, 8)
        base = pl.multiple_of(c * (2 * E) + s * EDGES_PER_TILE, 8)

        bufs = ((src_a, dst_a, sis_a, sid_a, rows_a, ss_a),
                (src_b, dst_b, sis_b, sid_b, rows_b, ss_b))

        def idx_start(j, p):
            sv, dv, ss, sd = bufs[p][:4]
            off = pl.multiple_of(base + j * CHUNK, 8)
            pltpu.async_copy(edges_hbm.at[pl.ds(off, CHUNK)], sv, ss)
            pltpu.async_copy(edges_hbm.at[pl.ds(off + E, CHUNK)], dv, sd)

        def idx_wait(j, p):
            sv, dv, ss, sd = bufs[p][:4]
            off = pl.multiple_of(base + j * CHUNK, 8)
            pltpu.make_async_copy(
                edges_hbm.at[pl.ds(off, CHUNK)], sv, ss).wait()
            pltpu.make_async_copy(
                edges_hbm.at[pl.ds(off + E, CHUNK)], dv, sd).wait()

        def scatter_start(p):
            _, dv, _, _, rv, sp = bufs[p]
            pltpu.async_copy(rv, acc.at[dv], sp, add=True)

        def scatter_wait(p):
            _, dv, _, _, rv, sp = bufs[p]
            pltpu.make_async_copy(rv, acc.at[dv], sp).wait()

        idx_start(0, 0)
        pltpu.sync_copy(x_hbm.at[pl.ds(r0, ROWS_PER_TILE)],
                        acc.at[pl.ds(r0, ROWS_PER_TILE)])

        @pl.when(s == NUM_TILES - 1)
        def _init_tail():
            t0 = NUM_TILES * ROWS_PER_TILE
            pltpu.sync_copy(x_hbm.at[pl.ds(t0, ROWS_TAIL)],
                            acc.at[pl.ds(t0, ROWS_TAIL)])

        plsc.subcore_barrier()

        def step(j, p, prefetch, drain=True):
            sv, _, _, _, rv, _ = bufs[p]
            idx_wait(j, p)
            pltpu.async_copy(x_hbm.at[sv], rv, sem).wait()
            if drain:
                scatter_wait(1 - p)
            if prefetch:
                idx_start(j + 1, 1 - p)
            scatter_start(p)

        step(0, 0, prefetch=True, drain=False)

        def body(jo, carry):
            step(jo * 2 + 1, 1, prefetch=True)
            step(jo * 2 + 2, 0, prefetch=True)
            return carry

        lax.fori_loop(0, NFULL // 2 - 1, body, 0)
        step(NFULL - 1, 1, prefetch=False)
        scatter_wait(1)

        off = base + NFULL * CHUNK
        pltpu.sync_copy(edges_hbm.at[pl.ds(off, REM)], src_r)
        pltpu.sync_copy(edges_hbm.at[pl.ds(off + E, REM)], dst_r)
        pltpu.async_copy(x_hbm.at[src_r], rows_r, sem).wait()
        pltpu.sync_copy(rows_r, acc.at[dst_r], add=True)

        plsc.subcore_barrier()
        pltpu.sync_copy(acc.at[pl.ds(r0, ROWS_PER_TILE)],
                        out_hbm.at[c, pl.ds(r0, ROWS_PER_TILE)])

        @pl.when(s == NUM_TILES - 1)
        def _out_tail():
            t0 = NUM_TILES * ROWS_PER_TILE
            pltpu.sync_copy(acc.at[pl.ds(t0, ROWS_TAIL)],
                            out_hbm.at[c, pl.ds(t0, ROWS_TAIL)])

    return agg_kernel(x, edges)


def _tc_mlp(pre, w1t0, w2t0, g10, b10, g20, b20, w1t1, w2t1, g11, b11, g21, b21):
    def body(pre_ref, w1t0_r, w2t0_r, g10_r, b10_r, g20_r, b20_r,
             w1t1_r, w2t1_r, g11_r, b11_r, g21_r, b21_r, out_ref):
        def bn_relu(h, g, b):
            mean = jnp.mean(h, axis=0, keepdims=True)
            var = jnp.mean((h - mean) * (h - mean), axis=0, keepdims=True)
            return jnp.maximum((h - mean) * lax.rsqrt(var + BN_EPS) * g + b, 0.0)

        def rel(p, w1t, w2t, g1, b1, g2, b2):
            h = jnp.dot(p, w1t, preferred_element_type=jnp.float32)
            h = bn_relu(h, g1, b1)
            h = jnp.dot(h, w2t, preferred_element_type=jnp.float32)
            return bn_relu(h, g2, b2)

        out_ref[...] = (
            rel(pre_ref[0], w1t0_r[...], w2t0_r[...], g10_r[...], b10_r[...],
                g20_r[...], b20_r[...])
            + rel(pre_ref[1], w1t1_r[...], w2t1_r[...], g11_r[...], b11_r[...],
                  g21_r[...], b21_r[...]))

    return pl.pallas_call(
        body,
        out_shape=jax.ShapeDtypeStruct((N, D), jnp.float32),
    )(pre, w1t0, w2t0, g10, b10, g20, b20, w1t1, w2t1, g11, b11, g21, b21)


@jax.jit
def kernel(x, edge_index_rel0, edge_index_rel1,
           W1_0, W2_0, g1_0, b1_0, g2_0, b2_0,
           W1_1, W2_1, g1_1, b1_1, g2_1, b2_1):
    edges = jnp.concatenate(
        [edge_index_rel0.reshape(-1), edge_index_rel1.reshape(-1)])
    pre = _sc_aggregate(x, edges)
    row = lambda v: v.reshape(1, D)
    return _tc_mlp(pre,
                   W1_0.T, W2_0.T, row(g1_0), row(b1_0), row(g2_0), row(b2_0),
                   W1_1.T, W2_1.T, row(g1_1), row(b1_1), row(g2_1), row(b2_1))
